# initial kernel scaffold (unmeasured)
import jax
import jax.numpy as jnp
from jax import lax
from jax.experimental import pallas as pl
from jax.experimental.pallas import tpu as pltpu

SCALE = 64 ** -0.5


def _body(q_ref, k_ref, v_ref, out_ref,
          acc_o, acc_m, acc_l, recv_o, recv_m, recv_l,
          send_sems, recv_sems):
    b = pl.program_id(0)
    nb = pl.num_programs(0)

    q = q_ref[0, 0]
    k = k_ref[0]
    v = v_ref[0]

    s = jnp.sum(k * q[None, :, :], axis=-1) * SCALE
    m = jnp.max(s, axis=0, keepdims=True)
    p = jnp.exp(s - m)
    l = jnp.sum(p, axis=0, keepdims=True)
    o = jnp.sum(p[:, :, None] * v, axis=0)

    acc_o[pl.ds(b, 1)] = o[None, :, :]
    acc_m[pl.ds(b, 1)] = m
    acc_l[pl.ds(b, 1)] = l

    @pl.when(b == nb - 1)
    def _():
        my_x = lax.axis_index("x")
        my_y = lax.axis_index("y")
        my_z = lax.axis_index("z")
        nbr = (my_x, 1 - my_y, my_z)

        barrier = pltpu.get_barrier_semaphore()
        pl.semaphore_signal(barrier, inc=1, device_id=nbr,
                            device_id_type=pl.DeviceIdType.MESH)
        pl.semaphore_wait(barrier, 1)

        rdmas = []
        for i, (src, dst) in enumerate(
            [(acc_o, recv_o), (acc_m, recv_m), (acc_l, recv_l)]
        ):
            r = pltpu.make_async_remote_copy(
                src_ref=src, dst_ref=dst,
                send_sem=send_sems.at[i], recv_sem=recv_sems.at[i],
                device_id=nbr, device_id_type=pl.DeviceIdType.MESH,
            )
            r.start()
            rdmas.append(r)
        for r in rdmas:
            r.wait()

        m_a = acc_m[:]
        l_a = acc_l[:]
        o_a = acc_o[:]
        m_b = recv_m[:]
        l_b = recv_l[:]
        o_b = recv_o[:]

        m_n = jnp.maximum(m_a, m_b)
        ea = jnp.exp(m_a - m_n)
        eb = jnp.exp(m_b - m_n)
        l_n = l_a * ea + l_b * eb
        o_n = (o_a * ea[:, :, None] + o_b * eb[:, :, None]) / l_n[:, :, None]
        out_ref[:] = o_n[:, None, :, :]


def kernel(Q, K, V):
    b, _, h, d = Q.shape
    kv = K.shape[1]
    return pl.pallas_call(
        _body,
        grid=(b,),
        in_specs=[
            pl.BlockSpec((1, 1, h, d), lambda i: (i, 0, 0, 0)),
            pl.BlockSpec((1, kv, h, d), lambda i: (i, 0, 0, 0)),
            pl.BlockSpec((1, kv, h, d), lambda i: (i, 0, 0, 0)),
        ],
        out_specs=pl.BlockSpec((b, 1, h, d), lambda i: (0, 0, 0, 0)),
        out_shape=jax.ShapeDtypeStruct((b, 1, h, d), jnp.float32),
        scratch_shapes=[
            pltpu.VMEM((b, h, d), jnp.float32),
            pltpu.VMEM((b, h), jnp.float32),
            pltpu.VMEM((b, h), jnp.float32),
            pltpu.VMEM((b, h, d), jnp.float32),
            pltpu.VMEM((b, h), jnp.float32),
            pltpu.VMEM((b, h), jnp.float32),
            pltpu.SemaphoreType.DMA((3,)),
            pltpu.SemaphoreType.DMA((3,)),
        ],
        compiler_params=pltpu.CompilerParams(
            collective_id=0,
            dimension_semantics=("arbitrary",),
        ),
    )(Q, K, V)


# baseline (device time: 331326 ns/iter reference)
import jax
import jax.numpy as jnp
from jax import lax
from jax.experimental import pallas as pl
from jax.experimental.pallas import tpu as pltpu

SCALE = 64 ** -0.5


def _body(q_ref, k_ref, v_ref, out_ref,
          acc_o, acc_m, acc_l, recv_o, recv_m, recv_l,
          send_sems, recv_sems):
    b = pl.program_id(0)
    nb = pl.num_programs(0)

    q = q_ref[0, 0]
    k = k_ref[0]
    v = v_ref[0]

    s = jnp.sum(k * q[None, :, :], axis=-1) * SCALE
    m = jnp.max(s, axis=0, keepdims=True)
    p = jnp.exp(s - m)
    l = jnp.sum(p, axis=0, keepdims=True)
    o = jnp.sum(p[:, :, None] * v, axis=0)

    acc_o[pl.ds(b, 1)] = o[None, :, :]
    acc_m[pl.ds(b, 1)] = m
    acc_l[pl.ds(b, 1)] = l

    @pl.when(b == nb - 1)
    def _():
        my_x = lax.axis_index("x")
        my_y = lax.axis_index("y")
        my_z = lax.axis_index("z")
        nbr = (my_x, 1 - my_y, my_z)

        barrier = pltpu.get_barrier_semaphore()
        pl.semaphore_signal(barrier, inc=1, device_id=nbr,
                            device_id_type=pl.DeviceIdType.MESH)
        pl.semaphore_wait(barrier, 1)

        rdmas = []
        for i, (src, dst) in enumerate(
            [(acc_o, recv_o), (acc_m, recv_m), (acc_l, recv_l)]
        ):
            r = pltpu.make_async_remote_copy(
                src_ref=src, dst_ref=dst,
                send_sem=send_sems.at[i], recv_sem=recv_sems.at[i],
                device_id=nbr, device_id_type=pl.DeviceIdType.MESH,
            )
            r.start()
            rdmas.append(r)
        for r in rdmas:
            r.wait()

        m_a = acc_m[:]
        l_a = acc_l[:]
        o_a = acc_o[:]
        m_b = recv_m[:]
        l_b = recv_l[:]
        o_b = recv_o[:]

        m_n = jnp.maximum(m_a, m_b)
        ea = jnp.exp(m_a - m_n)
        eb = jnp.exp(m_b - m_n)
        l_n = l_a * ea + l_b * eb
        o_n = (o_a * ea[:, :, None] + o_b * eb[:, :, None]) / l_n[:, :, None]
        out_ref[:] = o_n[:, None, :, :]


def kernel(Q, K, V):
    b, _, h, d = Q.shape
    kv = K.shape[1]
    return pl.pallas_call(
        _body,
        grid=(b,),
        in_specs=[
            pl.BlockSpec((1, 1, h, d), lambda i: (i, 0, 0, 0)),
            pl.BlockSpec((1, kv, h, d), lambda i: (i, 0, 0, 0)),
            pl.BlockSpec((1, kv, h, d), lambda i: (i, 0, 0, 0)),
        ],
        out_specs=pl.BlockSpec((b, 1, h, d), lambda i: (0, 0, 0, 0)),
        out_shape=jax.ShapeDtypeStruct((b, 1, h, d), jnp.float32),
        scratch_shapes=[
            pltpu.VMEM((b, h, d), jnp.float32),
            pltpu.VMEM((b, h), jnp.float32),
            pltpu.VMEM((b, h), jnp.float32),
            pltpu.VMEM((b, h, d), jnp.float32),
            pltpu.VMEM((b, h), jnp.float32),
            pltpu.VMEM((b, h), jnp.float32),
            pltpu.SemaphoreType.DMA((3,)),
            pltpu.SemaphoreType.DMA((3,)),
        ],
        compiler_params=pltpu.CompilerParams(
            collective_id=0,
            dimension_semantics=("arbitrary",),
            vmem_limit_bytes=100 * 1024 * 1024,
        ),
    )(Q, K, V)


# device time: 317109 ns/iter; 1.0448x vs baseline; 1.0448x over previous
import jax
import jax.numpy as jnp
from jax import lax
from jax.experimental import pallas as pl
from jax.experimental.pallas import tpu as pltpu

SCALE = 64 ** -0.5


def _body(q_ref, k_ref, v_ref, out_ref,
          acc_o, acc_m, acc_l, recv_o, recv_m, recv_l,
          send_sems, recv_sems):
    b = pl.program_id(0)
    nb = pl.num_programs(0)

    q = q_ref[0, 0] * SCALE
    k3 = k_ref[0]
    v3 = v_ref[0]

    kv, h, d = k3.shape
    s2 = (k3 * q[None, :, :]).reshape(kv * h, d)
    ones_col = jnp.ones((d, 1), jnp.float32)
    s_col = jnp.dot(s2, ones_col, preferred_element_type=jnp.float32)
    s3 = s_col.reshape(kv, h, 1)

    m = jnp.max(s3)
    p3 = jnp.exp(s3 - m)
    l3 = jnp.sum(p3, axis=0)
    o = jnp.sum(p3 * v3, axis=0)

    acc_o[pl.ds(b, 1)] = o[None]
    acc_m[pl.ds(b, 1)] = m.reshape(1, 1)
    acc_l[pl.ds(b, 1)] = l3[None]

    @pl.when(b == nb - 1)
    def _():
        my_x = lax.axis_index("x")
        my_y = lax.axis_index("y")
        my_z = lax.axis_index("z")
        nbr = (my_x, 1 - my_y, my_z)

        barrier = pltpu.get_barrier_semaphore()
        pl.semaphore_signal(barrier, inc=1, device_id=nbr,
                            device_id_type=pl.DeviceIdType.MESH)
        pl.semaphore_wait(barrier, 1)

        rdmas = []
        for i, (src, dst) in enumerate(
            [(acc_o, recv_o), (acc_m, recv_m), (acc_l, recv_l)]
        ):
            r = pltpu.make_async_remote_copy(
                src_ref=src, dst_ref=dst,
                send_sem=send_sems.at[i], recv_sem=recv_sems.at[i],
                device_id=nbr, device_id_type=pl.DeviceIdType.MESH,
            )
            r.start()
            rdmas.append(r)
        for r in rdmas:
            r.wait()

        m_a = acc_m[:]
        l_a = acc_l[:]
        o_a = acc_o[:]
        m_b = recv_m[:]
        l_b = recv_l[:]
        o_b = recv_o[:]

        m_n = jnp.maximum(m_a, m_b)
        ea = jnp.exp(m_a - m_n)[:, :, None]
        eb = jnp.exp(m_b - m_n)[:, :, None]
        l_n = l_a * ea + l_b * eb
        o_n = (o_a * ea + o_b * eb) / l_n
        out_ref[:] = o_n[:, None]


def kernel(Q, K, V):
    b, _, h, d = Q.shape
    kv = K.shape[1]
    return pl.pallas_call(
        _body,
        grid=(b,),
        in_specs=[
            pl.BlockSpec((1, 1, h, d), lambda i: (i, 0, 0, 0)),
            pl.BlockSpec((1, kv, h, d), lambda i: (i, 0, 0, 0)),
            pl.BlockSpec((1, kv, h, d), lambda i: (i, 0, 0, 0)),
        ],
        out_specs=pl.BlockSpec((b, 1, h, d), lambda i: (0, 0, 0, 0)),
        out_shape=jax.ShapeDtypeStruct((b, 1, h, d), jnp.float32),
        scratch_shapes=[
            pltpu.VMEM((b, h, d), jnp.float32),
            pltpu.VMEM((b, 1), jnp.float32),
            pltpu.VMEM((b, h, 1), jnp.float32),
            pltpu.VMEM((b, h, d), jnp.float32),
            pltpu.VMEM((b, 1), jnp.float32),
            pltpu.VMEM((b, h, 1), jnp.float32),
            pltpu.SemaphoreType.DMA((3,)),
            pltpu.SemaphoreType.DMA((3,)),
        ],
        compiler_params=pltpu.CompilerParams(
            collective_id=0,
            dimension_semantics=("arbitrary",),
            vmem_limit_bytes=100 * 1024 * 1024,
        ),
    )(Q, K, V)


# device time: 200110 ns/iter; 1.6557x vs baseline; 1.5847x over previous
import jax
import jax.numpy as jnp
from jax import lax
from jax.experimental import pallas as pl
from jax.experimental.pallas import tpu as pltpu

SCALE = 64 ** -0.5


def _selector(hd, h, dtype):
    d = hd // h
    row = lax.broadcasted_iota(jnp.int32, (hd, h), 0) // d
    col = lax.broadcasted_iota(jnp.int32, (hd, h), 1)
    return (row == col).astype(dtype)


def _body(q_ref, k_ref, v_ref, out_ref,
          acc_o, acc_m, acc_l, recv_o, recv_m, recv_l,
          send_sems, recv_sems):
    b = pl.program_id(0)
    nb = pl.num_programs(0)

    qcol = q_ref[0]
    kf = k_ref[0]
    vf = v_ref[0]
    kv, hd = kf.shape
    h = acc_m.shape[1]

    e = _selector(hd, h, kf.dtype)
    qe = (qcol * SCALE) * e
    s = jnp.dot(kf, qe, preferred_element_type=jnp.float32)
    m = jnp.max(s, axis=0, keepdims=True)
    p = jnp.exp(s - m)
    l = jnp.sum(p, axis=0, keepdims=True)
    px = jnp.dot(p, e.T, preferred_element_type=jnp.float32)
    ones_row = jnp.ones((1, kv), jnp.float32)
    o = jnp.dot(ones_row, px * vf, preferred_element_type=jnp.float32)

    acc_o[pl.ds(b, 1)] = o
    acc_m[pl.ds(b, 1)] = m
    acc_l[pl.ds(b, 1)] = l

    @pl.when(b == nb - 1)
    def _():
        my_x = lax.axis_index("x")
        my_y = lax.axis_index("y")
        my_z = lax.axis_index("z")
        nbr = (my_x, 1 - my_y, my_z)

        barrier = pltpu.get_barrier_semaphore()
        pl.semaphore_signal(barrier, inc=1, device_id=nbr,
                            device_id_type=pl.DeviceIdType.MESH)
        pl.semaphore_wait(barrier, 1)

        rdmas = []
        for i, (src, dst) in enumerate(
            [(acc_o, recv_o), (acc_m, recv_m), (acc_l, recv_l)]
        ):
            r = pltpu.make_async_remote_copy(
                src_ref=src, dst_ref=dst,
                send_sem=send_sems.at[i], recv_sem=recv_sems.at[i],
                device_id=nbr, device_id_type=pl.DeviceIdType.MESH,
            )
            r.start()
            rdmas.append(r)
        for r in rdmas:
            r.wait()

        m_a = acc_m[:]
        l_a = acc_l[:]
        o_a = acc_o[:]
        m_b = recv_m[:]
        l_b = recv_l[:]
        o_b = recv_o[:]

        m_n = jnp.maximum(m_a, m_b)
        ea = jnp.exp(m_a - m_n)
        eb = jnp.exp(m_b - m_n)
        l_n = l_a * ea + l_b * eb
        e = _selector(o_a.shape[1], m_a.shape[1], o_a.dtype)
        et = e.T
        ea_x = jnp.dot(ea, et, preferred_element_type=jnp.float32)
        eb_x = jnp.dot(eb, et, preferred_element_type=jnp.float32)
        li_x = jnp.dot(1.0 / l_n, et, preferred_element_type=jnp.float32)
        out_ref[:] = (o_a * ea_x + o_b * eb_x) * li_x


def kernel(Q, K, V):
    b, _, h, d = Q.shape
    kv = K.shape[1]
    hd = h * d
    qc = Q.reshape(b, hd, 1)
    kf = K.reshape(b, kv, hd)
    vf = V.reshape(b, kv, hd)
    out = pl.pallas_call(
        _body,
        grid=(b,),
        in_specs=[
            pl.BlockSpec((1, hd, 1), lambda i: (i, 0, 0)),
            pl.BlockSpec((1, kv, hd), lambda i: (i, 0, 0)),
            pl.BlockSpec((1, kv, hd), lambda i: (i, 0, 0)),
        ],
        out_specs=pl.BlockSpec((b, hd), lambda i: (0, 0)),
        out_shape=jax.ShapeDtypeStruct((b, hd), jnp.float32),
        scratch_shapes=[
            pltpu.VMEM((b, hd), jnp.float32),
            pltpu.VMEM((b, h), jnp.float32),
            pltpu.VMEM((b, h), jnp.float32),
            pltpu.VMEM((b, hd), jnp.float32),
            pltpu.VMEM((b, h), jnp.float32),
            pltpu.VMEM((b, h), jnp.float32),
            pltpu.SemaphoreType.DMA((3,)),
            pltpu.SemaphoreType.DMA((3,)),
        ],
        compiler_params=pltpu.CompilerParams(
            collective_id=0,
            dimension_semantics=("arbitrary",),
            vmem_limit_bytes=100 * 1024 * 1024,
        ),
    )(qc, kf, vf)
    return out.reshape(b, 1, h, d)


# device time: 51404 ns/iter; 6.4455x vs baseline; 3.8929x over previous
import jax
import jax.numpy as jnp
from jax import lax
from jax.experimental import pallas as pl
from jax.experimental.pallas import tpu as pltpu

SCALE = 64 ** -0.5


def _body(q_ref, k_ref, v_ref, out_ref,
          acc_o, acc_m, acc_l, recv_o, recv_m, recv_l,
          send_sems, recv_sems):
    b = pl.program_id(0)
    nb = pl.num_programs(0)

    q = q_ref[0, 0]
    k3 = k_ref[0]
    v3 = v_ref[0]

    s = jnp.sum(k3 * (q * SCALE)[:, :, None], axis=1)
    m = jnp.max(s, axis=1, keepdims=True)
    p = jnp.exp(s - m)
    l = jnp.sum(p, axis=1, keepdims=True)
    o = jnp.sum(p[:, None, :] * v3, axis=2)

    acc_o[pl.ds(b, 1)] = o[None]
    acc_m[pl.ds(b, 1)] = m[None]
    acc_l[pl.ds(b, 1)] = l[None]

    @pl.when(b == nb - 1)
    def _():
        my_x = lax.axis_index("x")
        my_y = lax.axis_index("y")
        my_z = lax.axis_index("z")
        nbr = (my_x, 1 - my_y, my_z)

        barrier = pltpu.get_barrier_semaphore()
        pl.semaphore_signal(barrier, inc=1, device_id=nbr,
                            device_id_type=pl.DeviceIdType.MESH)
        pl.semaphore_wait(barrier, 1)

        rdmas = []
        for i, (src, dst) in enumerate(
            [(acc_o, recv_o), (acc_m, recv_m), (acc_l, recv_l)]
        ):
            r = pltpu.make_async_remote_copy(
                src_ref=src, dst_ref=dst,
                send_sem=send_sems.at[i], recv_sem=recv_sems.at[i],
                device_id=nbr, device_id_type=pl.DeviceIdType.MESH,
            )
            r.start()
            rdmas.append(r)
        for r in rdmas:
            r.wait()

        m_a = acc_m[:]
        l_a = acc_l[:]
        o_a = acc_o[:]
        m_b = recv_m[:]
        l_b = recv_l[:]
        o_b = recv_o[:]

        m_n = jnp.maximum(m_a, m_b)
        ea = jnp.exp(m_a - m_n)
        eb = jnp.exp(m_b - m_n)
        l_n = l_a * ea + l_b * eb
        o_n = (o_a * ea + o_b * eb) / l_n
        out_ref[:] = o_n[:, None]


def kernel(Q, K, V):
    b, _, h, d = Q.shape
    kv = K.shape[1]
    kt = jnp.transpose(K, (0, 2, 3, 1))
    vt = jnp.transpose(V, (0, 2, 3, 1))
    return pl.pallas_call(
        _body,
        grid=(b,),
        in_specs=[
            pl.BlockSpec((1, 1, h, d), lambda i: (i, 0, 0, 0)),
            pl.BlockSpec((1, h, d, kv), lambda i: (i, 0, 0, 0)),
            pl.BlockSpec((1, h, d, kv), lambda i: (i, 0, 0, 0)),
        ],
        out_specs=pl.BlockSpec((b, 1, h, d), lambda i: (0, 0, 0, 0)),
        out_shape=jax.ShapeDtypeStruct((b, 1, h, d), jnp.float32),
        scratch_shapes=[
            pltpu.VMEM((b, h, d), jnp.float32),
            pltpu.VMEM((b, h, 1), jnp.float32),
            pltpu.VMEM((b, h, 1), jnp.float32),
            pltpu.VMEM((b, h, d), jnp.float32),
            pltpu.VMEM((b, h, 1), jnp.float32),
            pltpu.VMEM((b, h, 1), jnp.float32),
            pltpu.SemaphoreType.DMA((3,)),
            pltpu.SemaphoreType.DMA((3,)),
        ],
        compiler_params=pltpu.CompilerParams(
            collective_id=0,
            dimension_semantics=("arbitrary",),
            vmem_limit_bytes=100 * 1024 * 1024,
        ),
    )(Q, kt, vt)


# device time: 51235 ns/iter; 6.4668x vs baseline; 1.0033x over previous
import jax
import jax.numpy as jnp
from jax import lax
from jax.experimental import pallas as pl
from jax.experimental.pallas import tpu as pltpu

SCALE = 64 ** -0.5


def _body(q_ref, k_ref, v_ref, out_ref,
          acc_o, acc_m, acc_l, recv_o, recv_m, recv_l,
          send_sems, recv_sems):
    b = pl.program_id(0)
    nb = pl.num_programs(0)

    q = q_ref[0, 0]
    k3 = k_ref[0]
    v3 = v_ref[0]

    qs = (q * SCALE)[:, None, :]
    s3 = lax.dot_general(
        qs, k3, (((2,), (1,)), ((0,), (0,))),
        preferred_element_type=jnp.float32,
    )
    m = jnp.max(s3, axis=2, keepdims=True)
    p = jnp.exp(s3 - m)
    l = jnp.sum(p, axis=2, keepdims=True)
    o = jnp.sum(p * v3, axis=2)

    acc_o[pl.ds(b, 1)] = o[None]
    acc_m[pl.ds(b, 1)] = m[:, 0, :][None]
    acc_l[pl.ds(b, 1)] = l[:, 0, :][None]

    @pl.when(b == nb - 1)
    def _():
        my_x = lax.axis_index("x")
        my_y = lax.axis_index("y")
        my_z = lax.axis_index("z")
        nbr = (my_x, 1 - my_y, my_z)

        barrier = pltpu.get_barrier_semaphore()
        pl.semaphore_signal(barrier, inc=1, device_id=nbr,
                            device_id_type=pl.DeviceIdType.MESH)
        pl.semaphore_wait(barrier, 1)

        rdmas = []
        for i, (src, dst) in enumerate(
            [(acc_o, recv_o), (acc_m, recv_m), (acc_l, recv_l)]
        ):
            r = pltpu.make_async_remote_copy(
                src_ref=src, dst_ref=dst,
                send_sem=send_sems.at[i], recv_sem=recv_sems.at[i],
                device_id=nbr, device_id_type=pl.DeviceIdType.MESH,
            )
            r.start()
            rdmas.append(r)
        for r in rdmas:
            r.wait()

        m_a = acc_m[:]
        l_a = acc_l[:]
        o_a = acc_o[:]
        m_b = recv_m[:]
        l_b = recv_l[:]
        o_b = recv_o[:]

        m_n = jnp.maximum(m_a, m_b)
        ea = jnp.exp(m_a - m_n)
        eb = jnp.exp(m_b - m_n)
        l_n = l_a * ea + l_b * eb
        o_n = (o_a * ea + o_b * eb) / l_n
        out_ref[:] = o_n[:, None]


def kernel(Q, K, V):
    b, _, h, d = Q.shape
    kv = K.shape[1]
    kt = jnp.transpose(K, (0, 2, 3, 1))
    vt = jnp.transpose(V, (0, 2, 3, 1))
    return pl.pallas_call(
        _body,
        grid=(b,),
        in_specs=[
            pl.BlockSpec((1, 1, h, d), lambda i: (i, 0, 0, 0)),
            pl.BlockSpec((1, h, d, kv), lambda i: (i, 0, 0, 0)),
            pl.BlockSpec((1, h, d, kv), lambda i: (i, 0, 0, 0)),
        ],
        out_specs=pl.BlockSpec((b, 1, h, d), lambda i: (0, 0, 0, 0)),
        out_shape=jax.ShapeDtypeStruct((b, 1, h, d), jnp.float32),
        scratch_shapes=[
            pltpu.VMEM((b, h, d), jnp.float32),
            pltpu.VMEM((b, h, 1), jnp.float32),
            pltpu.VMEM((b, h, 1), jnp.float32),
            pltpu.VMEM((b, h, d), jnp.float32),
            pltpu.VMEM((b, h, 1), jnp.float32),
            pltpu.VMEM((b, h, 1), jnp.float32),
            pltpu.SemaphoreType.DMA((3,)),
            pltpu.SemaphoreType.DMA((3,)),
        ],
        compiler_params=pltpu.CompilerParams(
            collective_id=0,
            dimension_semantics=("arbitrary",),
            vmem_limit_bytes=100 * 1024 * 1024,
        ),
    )(Q, kt, vt)


# device time: 51192 ns/iter; 6.4722x vs baseline; 1.0008x over previous
import jax
import jax.numpy as jnp
from jax import lax
from jax.experimental import pallas as pl
from jax.experimental.pallas import tpu as pltpu

SCALE = 64 ** -0.5


def _body(q_ref, k_ref, v_ref, out_ref,
          acc_o, acc_m, acc_l, recv_o, recv_m, recv_l,
          send_sems, recv_sems):
    b = pl.program_id(0)
    nb = pl.num_programs(0)

    q = q_ref[0, 0]
    k3 = k_ref[0]
    v3 = v_ref[0]

    s = jnp.sum(k3 * (q * SCALE)[:, :, None], axis=1)
    m = jnp.max(s, axis=1, keepdims=True)
    p = jnp.exp(s - m)
    l = jnp.sum(p, axis=1, keepdims=True)
    o = jnp.sum(p[:, None, :] * v3, axis=2)

    acc_o[pl.ds(b, 1)] = o[None]
    acc_m[pl.ds(b, 1)] = m[None]
    acc_l[pl.ds(b, 1)] = l[None]

    my_x = lax.axis_index("x")
    my_y = lax.axis_index("y")
    my_z = lax.axis_index("z")
    nbr = (my_x, 1 - my_y, my_z)

    @pl.when(b == 0)
    def _():
        barrier = pltpu.get_barrier_semaphore()
        pl.semaphore_signal(barrier, inc=1, device_id=nbr,
                            device_id_type=pl.DeviceIdType.MESH)
        pl.semaphore_wait(barrier, 1)

    @pl.when(b == nb - 1)
    def _():
        rdmas = []
        for i, (src, dst) in enumerate(
            [(acc_o, recv_o), (acc_m, recv_m), (acc_l, recv_l)]
        ):
            r = pltpu.make_async_remote_copy(
                src_ref=src, dst_ref=dst,
                send_sem=send_sems.at[i], recv_sem=recv_sems.at[i],
                device_id=nbr, device_id_type=pl.DeviceIdType.MESH,
            )
            r.start()
            rdmas.append(r)
        for r in rdmas:
            r.wait()

        m_a = acc_m[:]
        l_a = acc_l[:]
        o_a = acc_o[:]
        m_b = recv_m[:]
        l_b = recv_l[:]
        o_b = recv_o[:]

        m_n = jnp.maximum(m_a, m_b)
        ea = jnp.exp(m_a - m_n)
        eb = jnp.exp(m_b - m_n)
        l_n = l_a * ea + l_b * eb
        o_n = (o_a * ea + o_b * eb) / l_n
        out_ref[:] = o_n[:, None]


def kernel(Q, K, V):
    b, _, h, d = Q.shape
    kv = K.shape[1]
    kt = jnp.transpose(K, (0, 2, 3, 1))
    vt = jnp.transpose(V, (0, 2, 3, 1))
    return pl.pallas_call(
        _body,
        grid=(b,),
        in_specs=[
            pl.BlockSpec((1, 1, h, d), lambda i: (i, 0, 0, 0)),
            pl.BlockSpec((1, h, d, kv), lambda i: (i, 0, 0, 0)),
            pl.BlockSpec((1, h, d, kv), lambda i: (i, 0, 0, 0)),
        ],
        out_specs=pl.BlockSpec((b, 1, h, d), lambda i: (0, 0, 0, 0)),
        out_shape=jax.ShapeDtypeStruct((b, 1, h, d), jnp.float32),
        scratch_shapes=[
            pltpu.VMEM((b, h, d), jnp.float32),
            pltpu.VMEM((b, h, 1), jnp.float32),
            pltpu.VMEM((b, h, 1), jnp.float32),
            pltpu.VMEM((b, h, d), jnp.float32),
            pltpu.VMEM((b, h, 1), jnp.float32),
            pltpu.VMEM((b, h, 1), jnp.float32),
            pltpu.SemaphoreType.DMA((3,)),
            pltpu.SemaphoreType.DMA((3,)),
        ],
        compiler_params=pltpu.CompilerParams(
            collective_id=0,
            dimension_semantics=("arbitrary",),
            vmem_limit_bytes=100 * 1024 * 1024,
        ),
    )(Q, kt, vt)
